# baseline (speedup 1.0000x reference)
import functools

import jax
import jax.numpy as jnp
from jax.experimental import pallas as pl
from jax.experimental.pallas import tpu as pltpu

_BM = 200


_ISSUE_STEPS = 8
_DRAIN_STEP = 9


def _acc_schedule(n, bm, k_steps):
    bounds = [0]
    for f in (0.25, 0.5, 0.75):
        b = (int(n * f) // 128) * 128
        if b > bounds[-1]:
            bounds.append(b)
    last_aligned = ((n - bm) // 128) * 128
    if last_aligned > bounds[-1]:
        bounds.append(last_aligned)
    chunks = []
    for lo, hi in zip(bounds[:-1], bounds[1:]):
        step = max(-(-hi // bm) - 1, _DRAIN_STEP + 1)
        assert (step + 1) * bm >= hi and step < k_steps - 1
        chunks.append((step, lo, hi))
    return chunks, bounds[-1]


def _gcn_body(idx_ref,
              x_ref, w1_ref, b1_ref, w2_ref, b2_ref, fn_ref, nf_ref,
              out_ref,
              s1_ref, s2_ref, nfg_ref, acc_ref, sem,
              *, k_steps, bm, nidx, n):
    i = pl.program_id(0)
    per_issue = nidx // _ISSUE_STEPS

    @pl.when(i == 0)
    def _prologue():
        s1_ref[...] = jnp.dot(x_ref[...], w1_ref[...],
                              preferred_element_type=jnp.float32)

    @pl.when(i < _ISSUE_STEPS)
    def _issue_gather():
        def issue(j, carry):
            r = i * per_issue + j
            pltpu.make_async_copy(nf_ref.at[idx_ref[r]], nfg_ref.at[r],
                                  sem).start()
            return carry
        jax.lax.fori_loop(0, per_issue, issue, 0)

    h = jnp.dot(fn_ref[...], s1_ref[...], preferred_element_type=jnp.float32)
    h = jnp.maximum(h + b1_ref[...], 0.0)
    s2_blk = jnp.dot(h, w2_ref[...], preferred_element_type=jnp.float32)
    s2_ref[pl.ds(i * bm, bm), :] = s2_blk

    @pl.when(i == _DRAIN_STEP)
    def _drain():
        def wait(r, carry):
            pltpu.make_async_copy(nf_ref.at[idx_ref[r]], nfg_ref.at[r],
                                  sem).wait()
            return carry
        jax.lax.fori_loop(0, nidx, wait, 0)

    chunks, tail_lo = _acc_schedule(n, bm, k_steps)
    for _ci, (_step, _lo, _hi) in enumerate(chunks):
        @pl.when(i == _step)
        def _acc_chunk(_ci=_ci, _lo=_lo, _hi=_hi):
            part = jnp.dot(nfg_ref[:, _lo:_hi], s2_ref[_lo:_hi, :],
                           preferred_element_type=jnp.float32)
            if _ci == 0:
                acc_ref[...] = part
            else:
                acc_ref[...] += part

    @pl.when(i == k_steps - 1)
    def _epilogue():
        o = acc_ref[...] + jnp.dot(nfg_ref[:, tail_lo:], s2_ref[tail_lo:, :],
                                   preferred_element_type=jnp.float32)
        o = o + b2_ref[...]
        o = o - jnp.max(o, axis=1, keepdims=True)
        e = jnp.exp(o)
        out_ref[...] = e / jnp.sum(e, axis=1, keepdims=True)


def kernel(x, NF, FN, idx, W1, b1, W2, b2):
    n, nfeat = x.shape
    nhid = W1.shape[1]
    nclass = W2.shape[1]
    nidx = idx.shape[0]
    k_steps = n // _BM

    grid_spec = pltpu.PrefetchScalarGridSpec(
        num_scalar_prefetch=1,
        grid=(k_steps,),
        in_specs=[
            pl.BlockSpec((n, nfeat), lambda i, idx_ref: (0, 0)),
            pl.BlockSpec((nfeat, nhid), lambda i, idx_ref: (0, 0)),
            pl.BlockSpec((1, nhid), lambda i, idx_ref: (0, 0)),
            pl.BlockSpec((nhid, nclass), lambda i, idx_ref: (0, 0)),
            pl.BlockSpec((1, nclass), lambda i, idx_ref: (0, 0)),
            pl.BlockSpec((_BM, n), lambda i, idx_ref: (i, 0)),
            pl.BlockSpec(memory_space=pltpu.MemorySpace.HBM),
        ],
        out_specs=pl.BlockSpec((nidx, nclass), lambda i, idx_ref: (0, 0)),
        scratch_shapes=[
            pltpu.VMEM((n, nhid), jnp.float32),
            pltpu.VMEM((n, nclass), jnp.float32),
            pltpu.VMEM((nidx, n), jnp.float32),
            pltpu.VMEM((nidx, nclass), jnp.float32),
            pltpu.SemaphoreType.DMA,
        ],
    )
    body = functools.partial(_gcn_body, k_steps=k_steps, bm=_BM, nidx=nidx,
                             n=n)
    return pl.pallas_call(
        body,
        grid_spec=grid_spec,
        out_shape=jax.ShapeDtypeStruct((nidx, nclass), jnp.float32),
        compiler_params=pltpu.CompilerParams(
            dimension_semantics=("arbitrary",)),
    )(idx.astype(jnp.int32), x, W1, b1.reshape(1, -1), W2, b2.reshape(1, -1),
      FN, NF)


# speedup vs baseline: 1.0242x; 1.0242x over previous
import functools

import jax
import jax.numpy as jnp
from jax.experimental import pallas as pl
from jax.experimental.pallas import tpu as pltpu

_BM = 400


_ISSUE_STEPS = 8
_DRAIN_STEP = 9


def _gcn_body(idx_ref,
              x_ref, w1_ref, b1_ref, w2_ref, b2_ref, fn_ref, nf_ref,
              out_ref,
              s1_ref, s2_ref, nfg_ref, sem,
              *, k_steps, bm, nidx, n):
    i = pl.program_id(0)
    per_issue = nidx // _ISSUE_STEPS

    @pl.when(i == 0)
    def _prologue():
        s1_ref[...] = jnp.dot(x_ref[...], w1_ref[...],
                              preferred_element_type=jnp.float32)

    @pl.when(i < _ISSUE_STEPS)
    def _issue_gather():
        def issue(j, carry):
            r = i * per_issue + j
            pltpu.make_async_copy(nf_ref.at[idx_ref[r]], nfg_ref.at[r],
                                  sem).start()
            return carry
        jax.lax.fori_loop(0, per_issue, issue, 0)

    h = jnp.dot(fn_ref[...], s1_ref[...], preferred_element_type=jnp.float32)
    h = jnp.maximum(h + b1_ref[...], 0.0)
    s2_blk = jnp.dot(h, w2_ref[...], preferred_element_type=jnp.float32)
    s2_ref[pl.ds(i * bm, bm), :] = s2_blk

    @pl.when(i == _DRAIN_STEP)
    def _drain():
        def wait(r, carry):
            pltpu.make_async_copy(nf_ref.at[idx_ref[r]], nfg_ref.at[r],
                                  sem).wait()
            return carry
        jax.lax.fori_loop(0, nidx, wait, 0)

    @pl.when(i == k_steps - 1)
    def _epilogue():
        o = jnp.dot(nfg_ref[...], s2_ref[...],
                    preferred_element_type=jnp.float32) + b2_ref[...]
        o = o - jnp.max(o, axis=1, keepdims=True)
        e = jnp.exp(o)
        out_ref[...] = e / jnp.sum(e, axis=1, keepdims=True)


def kernel(x, NF, FN, idx, W1, b1, W2, b2):
    n, nfeat = x.shape
    nhid = W1.shape[1]
    nclass = W2.shape[1]
    nidx = idx.shape[0]
    k_steps = n // _BM

    grid_spec = pltpu.PrefetchScalarGridSpec(
        num_scalar_prefetch=1,
        grid=(k_steps,),
        in_specs=[
            pl.BlockSpec((n, nfeat), lambda i, idx_ref: (0, 0)),
            pl.BlockSpec((nfeat, nhid), lambda i, idx_ref: (0, 0)),
            pl.BlockSpec((1, nhid), lambda i, idx_ref: (0, 0)),
            pl.BlockSpec((nhid, nclass), lambda i, idx_ref: (0, 0)),
            pl.BlockSpec((1, nclass), lambda i, idx_ref: (0, 0)),
            pl.BlockSpec((_BM, n), lambda i, idx_ref: (i, 0)),
            pl.BlockSpec(memory_space=pltpu.MemorySpace.HBM),
        ],
        out_specs=pl.BlockSpec((nidx, nclass), lambda i, idx_ref: (0, 0)),
        scratch_shapes=[
            pltpu.VMEM((n, nhid), jnp.float32),
            pltpu.VMEM((n, nclass), jnp.float32),
            pltpu.VMEM((nidx, n), jnp.float32),
            pltpu.SemaphoreType.DMA,
        ],
    )
    body = functools.partial(_gcn_body, k_steps=k_steps, bm=_BM, nidx=nidx,
                             n=n)
    return pl.pallas_call(
        body,
        grid_spec=grid_spec,
        out_shape=jax.ShapeDtypeStruct((nidx, nclass), jnp.float32),
        compiler_params=pltpu.CompilerParams(
            dimension_semantics=("arbitrary",)),
    )(idx.astype(jnp.int32), x, W1, b1.reshape(1, -1), W2, b2.reshape(1, -1),
      FN, NF)


# speedup vs baseline: 1.0360x; 1.0115x over previous
import functools

import jax
import jax.numpy as jnp
from jax.experimental import pallas as pl
from jax.experimental.pallas import tpu as pltpu

_BM = 400


_ISSUE_STEPS = 8
_DRAIN_STEP = 9


def _gcn_body(idx_ref,
              x_ref, w1_ref, b1_ref, w2_ref, b2_ref, fn_ref, nf_ref,
              out_ref,
              s1_ref, s2_ref, nfg_ref, acc_ref, sem,
              *, k_steps, bm, nidx, n):
    split = n - bm
    assert split % 128 == 0
    i = pl.program_id(0)
    per_issue = nidx // _ISSUE_STEPS

    @pl.when(i == 0)
    def _prologue():
        s1_ref[...] = jnp.dot(x_ref[...], w1_ref[...],
                              preferred_element_type=jnp.float32)

    @pl.when(i < _ISSUE_STEPS)
    def _issue_gather():
        def issue(j, carry):
            r = i * per_issue + j
            pltpu.make_async_copy(nf_ref.at[idx_ref[r]], nfg_ref.at[r],
                                  sem).start()
            return carry
        jax.lax.fori_loop(0, per_issue, issue, 0)

    h = jnp.dot(fn_ref[...], s1_ref[...], preferred_element_type=jnp.float32)
    h = jnp.maximum(h + b1_ref[...], 0.0)
    s2_blk = jnp.dot(h, w2_ref[...], preferred_element_type=jnp.float32)
    s2_ref[pl.ds(i * bm, bm), :] = s2_blk

    @pl.when(i == _DRAIN_STEP)
    def _drain():
        def wait(r, carry):
            pltpu.make_async_copy(nf_ref.at[idx_ref[r]], nfg_ref.at[r],
                                  sem).wait()
            return carry
        jax.lax.fori_loop(0, nidx, wait, 0)

    @pl.when(i == k_steps - 2)
    def _acc_main():
        acc_ref[...] = jnp.dot(nfg_ref[:, :split], s2_ref[:split, :],
                               preferred_element_type=jnp.float32)

    @pl.when(i == k_steps - 1)
    def _epilogue():
        o = acc_ref[...] + jnp.dot(nfg_ref[:, split:], s2_blk,
                                   preferred_element_type=jnp.float32)
        o = o + b2_ref[...]
        o = o - jnp.max(o, axis=1, keepdims=True)
        e = jnp.exp(o)
        out_ref[...] = e / jnp.sum(e, axis=1, keepdims=True)


def kernel(x, NF, FN, idx, W1, b1, W2, b2):
    n, nfeat = x.shape
    nhid = W1.shape[1]
    nclass = W2.shape[1]
    nidx = idx.shape[0]
    k_steps = n // _BM

    grid_spec = pltpu.PrefetchScalarGridSpec(
        num_scalar_prefetch=1,
        grid=(k_steps,),
        in_specs=[
            pl.BlockSpec((n, nfeat), lambda i, idx_ref: (0, 0)),
            pl.BlockSpec((nfeat, nhid), lambda i, idx_ref: (0, 0)),
            pl.BlockSpec((1, nhid), lambda i, idx_ref: (0, 0)),
            pl.BlockSpec((nhid, nclass), lambda i, idx_ref: (0, 0)),
            pl.BlockSpec((1, nclass), lambda i, idx_ref: (0, 0)),
            pl.BlockSpec((_BM, n), lambda i, idx_ref: (i, 0)),
            pl.BlockSpec(memory_space=pltpu.MemorySpace.HBM),
        ],
        out_specs=pl.BlockSpec((nidx, nclass), lambda i, idx_ref: (0, 0)),
        scratch_shapes=[
            pltpu.VMEM((n, nhid), jnp.float32),
            pltpu.VMEM((n, nclass), jnp.float32),
            pltpu.VMEM((nidx, n), jnp.float32),
            pltpu.VMEM((nidx, nclass), jnp.float32),
            pltpu.SemaphoreType.DMA,
        ],
    )
    body = functools.partial(_gcn_body, k_steps=k_steps, bm=_BM, nidx=nidx,
                             n=n)
    return pl.pallas_call(
        body,
        grid_spec=grid_spec,
        out_shape=jax.ShapeDtypeStruct((nidx, nclass), jnp.float32),
        compiler_params=pltpu.CompilerParams(
            dimension_semantics=("arbitrary",)),
    )(idx.astype(jnp.int32), x, W1, b1.reshape(1, -1), W2, b2.reshape(1, -1),
      FN, NF)
